# baseline (speedup 1.0000x reference)
import functools

import jax
import jax.numpy as jnp
from jax import lax
from jax.experimental import pallas as pl
from jax.experimental.pallas import tpu as pltpu
from jax.experimental.pallas import tpu_sc as plsc

L = 16
KSEL = 5
NSUB = 16
TPB = 8
BLK = 32768
NBLK = 16
PAD = 128


def _insert(ms, x):
    out = []
    for m in ms:
        lo = jnp.minimum(m, x)
        x = jnp.maximum(m, x)
        out.append(lo)
    return tuple(out)


SUBV = 128
NSUB_BLK = BLK // (SUBV * L)
NACC = 8
STEPV = 32


def _scan_buf(buf, ms):
    inf_v = jnp.full((L,), jnp.inf, jnp.float32)

    def sub(u, ms):
        base = pl.multiple_of(u * (SUBV * L), SUBV * L)

        def step(q, accs):
            o = base + q * (STEPV * L)
            accs = list(accs)
            for k in range(STEPV):
                accs[k % NACC] = jnp.minimum(
                    accs[k % NACC], buf[pl.ds(o + k * L, L)]
                )
            return tuple(accs)

        accs = lax.fori_loop(0, SUBV // STEPV, step, (inf_v,) * NACC)
        bm = accs[0]
        for a in range(1, NACC):
            bm = jnp.minimum(bm, accs[a])
        need = jnp.any(bm < ms[KSEL - 1])

        def rescan(ms):
            def ins(k, ms):
                return _insert(ms, buf[pl.ds(base + k * L, L)])

            return lax.fori_loop(0, SUBV, ins, ms)

        return lax.cond(need, rescan, lambda m: m, ms)

    return lax.fori_loop(0, NSUB_BLK, sub, ms)


def _sc_body(w_hbm, out_hbm, buf0, buf1, cand, gath, outv, shared,
             sem0, sem1):
    c = lax.axis_index("c")
    s = lax.axis_index("s")
    b = c * 2 + s // TPB
    p = s % TPB

    inf_v = jnp.full((L,), jnp.inf, jnp.float32)
    init = (inf_v,) * KSEL

    def dma(i, buf, sem):
        return pltpu.make_async_copy(w_hbm.at[b, p, i], buf, sem)

    dma(0, buf0, sem0).start()

    def pair_body(g, ms):
        i0 = 2 * g
        dma(i0 + 1, buf1, sem1).start()
        dma(i0, buf0, sem0).wait()
        ms = _scan_buf(buf0, ms)

        @pl.when(g < NBLK // 2 - 1)
        def _():
            dma(i0 + 2, buf0, sem0).start()

        dma(i0 + 1, buf1, sem1).wait()
        return _scan_buf(buf1, ms)

    ms = lax.fori_loop(0, NBLK // 2, pair_body, init)

    for j in range(KSEL):
        cand[pl.ds(j * L, L)] = ms[j]
    pltpu.sync_copy(cand, shared.at[s])
    plsc.subcore_barrier()

    @pl.when(p == 0)
    def _stage2():
        base = (s // TPB) * TPB
        pltpu.sync_copy(shared.at[pl.ds(base, TPB)], gath)
        cs = list(init)
        for t in range(TPB):
            for j in range(KSEL):
                cs = list(_insert(cs, gath[t, pl.ds(j * L, L)]))
        def xsort(v):
            return plsc.sort_key_val(v, v)[0]

        acc = xsort(cs[0])
        for j in range(1, KSEL):
            lo = jnp.minimum(acc, lax.rev(xsort(cs[j]), (0,)))
            acc = xsort(lo)
        outv[...] = plsc.cumsum(acc)
        pltpu.sync_copy(outv, out_hbm.at[b])


@jax.jit
def kernel(weights):
    batch = weights.shape[0]
    w = weights.reshape(batch, TPB, NBLK, BLK)
    mesh = plsc.VectorSubcoreMesh(core_axis_name="c", subcore_axis_name="s")
    run = pl.kernel(
        _sc_body,
        out_type=jax.ShapeDtypeStruct((batch, L), jnp.float32),
        mesh=mesh,
        compiler_params=pltpu.CompilerParams(needs_layout_passes=False),
        scratch_types=[
            pltpu.VMEM((BLK,), jnp.float32),
            pltpu.VMEM((BLK,), jnp.float32),
            pltpu.VMEM((PAD,), jnp.float32),
            pltpu.VMEM((TPB, PAD), jnp.float32),
            pltpu.VMEM((L,), jnp.float32),
            pltpu.VMEM_SHARED((NSUB, PAD), jnp.float32),
            pltpu.SemaphoreType.DMA,
            pltpu.SemaphoreType.DMA,
        ],
    )
    sums = run(w)
    return sums[:, KSEL - 1].sum() / batch


# speedup vs baseline: 1.3294x; 1.3294x over previous
import functools

import jax
import jax.numpy as jnp
from jax import lax
from jax.experimental import pallas as pl
from jax.experimental.pallas import tpu as pltpu
from jax.experimental.pallas import tpu_sc as plsc

L = 16
KSEL = 5
NSUB = 16
TPB = 8
BLK = 32768
NBLK = 16
PAD = 128


def _insert(ms, x):
    out = []
    for m in ms:
        lo = jnp.minimum(m, x)
        x = jnp.maximum(m, x)
        out.append(lo)
    return tuple(out)


SUBV = 64
NSUB_BLK = BLK // (SUBV * L)
NACC = 8
STEPV = 32


def _scan_buf(buf, ms):
    inf_v = jnp.full((L,), jnp.inf, jnp.float32)

    def sub(u, ms):
        base = pl.multiple_of(u * (SUBV * L), SUBV * L)

        def step(q, accs):
            o = base + q * (STEPV * L)
            accs = list(accs)
            for k in range(STEPV):
                accs[k % NACC] = jnp.minimum(
                    accs[k % NACC], buf[pl.ds(o + k * L, L)]
                )
            return tuple(accs)

        accs = lax.fori_loop(0, SUBV // STEPV, step, (inf_v,) * NACC)
        bm = accs[0]
        for a in range(1, NACC):
            bm = jnp.minimum(bm, accs[a])
        need = jnp.any(bm < ms[KSEL - 1])

        def rescan(ms):
            for k in range(SUBV):
                ms = _insert(ms, buf[pl.ds(base + k * L, L)])
            return ms

        return lax.cond(need, rescan, lambda m: m, ms)

    return lax.fori_loop(0, NSUB_BLK, sub, ms)


def _sc_body(w_hbm, out_hbm, buf0, buf1, cand, gath, outv, shared,
             sem0, sem1):
    c = lax.axis_index("c")
    s = lax.axis_index("s")
    b = c * 2 + s // TPB
    p = s % TPB

    inf_v = jnp.full((L,), jnp.inf, jnp.float32)
    init = (inf_v,) * KSEL

    def dma(i, buf, sem):
        return pltpu.make_async_copy(w_hbm.at[b, p, i], buf, sem)

    dma(0, buf0, sem0).start()

    def pair_body(g, ms):
        i0 = 2 * g
        dma(i0 + 1, buf1, sem1).start()
        dma(i0, buf0, sem0).wait()
        ms = _scan_buf(buf0, ms)

        @pl.when(g < NBLK // 2 - 1)
        def _():
            dma(i0 + 2, buf0, sem0).start()

        dma(i0 + 1, buf1, sem1).wait()
        return _scan_buf(buf1, ms)

    ms = lax.fori_loop(0, NBLK // 2, pair_body, init)

    for j in range(KSEL):
        cand[pl.ds(j * L, L)] = ms[j]
    pltpu.sync_copy(cand, shared.at[s])
    plsc.subcore_barrier()

    @pl.when(p == 0)
    def _stage2():
        base = (s // TPB) * TPB
        pltpu.sync_copy(shared.at[pl.ds(base, TPB)], gath)
        cs = list(init)
        for t in range(TPB):
            for j in range(KSEL):
                cs = list(_insert(cs, gath[t, pl.ds(j * L, L)]))
        def xsort(v):
            return plsc.sort_key_val(v, v)[0]

        acc = xsort(cs[0])
        for j in range(1, KSEL):
            lo = jnp.minimum(acc, lax.rev(xsort(cs[j]), (0,)))
            acc = xsort(lo)
        outv[...] = plsc.cumsum(acc)
        pltpu.sync_copy(outv, out_hbm.at[b])


@jax.jit
def kernel(weights):
    batch = weights.shape[0]
    w = weights.reshape(batch, TPB, NBLK, BLK)
    mesh = plsc.VectorSubcoreMesh(core_axis_name="c", subcore_axis_name="s")
    run = pl.kernel(
        _sc_body,
        out_type=jax.ShapeDtypeStruct((batch, L), jnp.float32),
        mesh=mesh,
        compiler_params=pltpu.CompilerParams(needs_layout_passes=False),
        scratch_types=[
            pltpu.VMEM((BLK,), jnp.float32),
            pltpu.VMEM((BLK,), jnp.float32),
            pltpu.VMEM((PAD,), jnp.float32),
            pltpu.VMEM((TPB, PAD), jnp.float32),
            pltpu.VMEM((L,), jnp.float32),
            pltpu.VMEM_SHARED((NSUB, PAD), jnp.float32),
            pltpu.SemaphoreType.DMA,
            pltpu.SemaphoreType.DMA,
        ],
    )
    sums = run(w)
    return sums[:, KSEL - 1].sum() / batch
